# Initial kernel scaffold; baseline (speedup 1.0000x reference)
import jax
import jax.numpy as jnp
from jax.experimental import pallas as pl


def kernel(x, edge_index, batch, metadata, W_enc, b_enc, W0, b0, g0, be0, W1, b1, g1, be1, Wc1, bc1, Wc2, bc2):
    raise NotImplementedError("write your pallas kernel here")


# speedup vs baseline: 10.1369x; 10.1369x over previous
import functools
import math
import jax
import jax.numpy as jnp
from jax import lax
from jax.experimental import pallas as pl
from jax.experimental.pallas import tpu as pltpu
from jax.experimental.pallas import tpu_sc as plsc

N = 10000
D = 128
G = 64
MD = 16
EPS = 1e-5

NPAD = 10240
DUMP = NPAD - 1
K = 128
NW = 32
CHW = 160
EPADDED = NW * CHW * K
ROWS_PER_SUB = NPAD // 16
CH_PER_SUB = ROWS_PER_SUB // K
BLK = 32

def _deg_body(dst_hbm, out_hbm, dstbuf, hist):
    c = lax.axis_index("c")
    s = lax.axis_index("s")
    wid = c * 16 + s
    z16 = jnp.zeros((16,), jnp.float32)
    ones16 = jnp.ones((16,), jnp.float32)

    def zero_body(i, _):
        hist[pl.ds(i * 16, 16)] = z16
        return 0

    lax.fori_loop(0, NPAD // 16, zero_body, 0)
    pltpu.sync_copy(dst_hbm.at[pl.ds(wid * CHW, CHW)], dstbuf)

    def body(j, _):
        for k in range(K // 16):
            idx = dstbuf[j, pl.ds(k * 16, 16)]
            plsc.addupdate_scatter(hist, [idx], ones16)
        return 0

    lax.fori_loop(0, CHW, body, 0)
    pltpu.sync_copy(hist, out_hbm.at[pl.ds(wid * NPAD, NPAD)])


def _scatter_body(u_hbm, src_hbm, dst_hbm, out_hbm, srcbuf, dstbuf, bufa,
                  bufb, acc, sema, semb):
    c = lax.axis_index("c")
    s = lax.axis_index("s")
    wid = c * 16 + s
    z16 = jnp.zeros((16,), jnp.float32)

    def zrow(i, _):
        for k in range(D // 16):
            bufa[i, pl.ds(k * 16, 16)] = z16
        return 0

    lax.fori_loop(0, K, zrow, 0)

    def zacc(i, _):
        pltpu.sync_copy(bufa, acc.at[pl.ds(s * ROWS_PER_SUB + i * K, K)])
        return 0

    lax.fori_loop(0, CH_PER_SUB, zacc, 0)
    plsc.subcore_barrier()

    bufs = (bufa, bufb)
    sems = (sema, semb)

    def block(bi, _):
        base = wid * CHW + bi * BLK
        pltpu.sync_copy(src_hbm.at[pl.ds(base, BLK)], srcbuf)
        pltpu.sync_copy(dst_hbm.at[pl.ds(base, BLK)], dstbuf)
        pltpu.async_copy(u_hbm.at[srcbuf.at[0]], bufa, sema)
        pltpu.async_copy(u_hbm.at[srcbuf.at[1]], bufb, semb)

        def body(i, _):
            for b in range(2):
                j = i * 2 + b
                pltpu.make_async_copy(u_hbm.at[srcbuf.at[j]], bufs[b],
                                      sems[b]).wait()
                pltpu.sync_copy(bufs[b], acc.at[dstbuf.at[j]], add=True)

                @pl.when(j + 2 < BLK)
                def _start():
                    pltpu.async_copy(u_hbm.at[srcbuf.at[j + 2]], bufs[b],
                                     sems[b])
            return 0

        lax.fori_loop(0, BLK // 2, body, 0)
        return 0

    lax.fori_loop(0, CHW // BLK, block, 0)
    plsc.subcore_barrier()

    def wb(i, _):
        r = s * ROWS_PER_SUB + i * K
        pltpu.sync_copy(acc.at[pl.ds(r, K)], bufa)
        pltpu.sync_copy(bufa, out_hbm.at[c, pl.ds(r, K)])
        return 0

    lax.fori_loop(0, CH_PER_SUB, wb, 0)


@functools.cache
def _sc_kernels():
    mesh = plsc.VectorSubcoreMesh(core_axis_name="c", subcore_axis_name="s",
                                  num_cores=2, num_subcores=16)
    deg = pl.kernel(
        _deg_body,
        out_type=jax.ShapeDtypeStruct((NW * NPAD,), jnp.float32),
        mesh=mesh,
        compiler_params=pltpu.CompilerParams(needs_layout_passes=False),
        scratch_types=[
            pltpu.VMEM((CHW, K), jnp.int32),
            pltpu.VMEM((NPAD,), jnp.float32),
        ],
    )
    scatter = pl.kernel(
        _scatter_body,
        out_type=jax.ShapeDtypeStruct((2, NPAD, D), jnp.float32),
        mesh=mesh,
        compiler_params=pltpu.CompilerParams(needs_layout_passes=False),
        scratch_types=[
            pltpu.VMEM((BLK, K), jnp.int32),
            pltpu.VMEM((BLK, K), jnp.int32),
            pltpu.VMEM((K, D), jnp.float32),
            pltpu.VMEM((K, D), jnp.float32),
            pltpu.VMEM_SHARED((NPAD, D), jnp.float32),
            pltpu.SemaphoreType.DMA,
            pltpu.SemaphoreType.DMA,
        ],
    )
    return deg, scatter


_R = 1000
_GRID = N // _R
_INV_S = 1.0 / math.sqrt(1.0 + EPS)


def _tc1_body(x_ref, we_ref, be_ref, w0_ref, dp_ref, u0_ref):
    h = jnp.dot(x_ref[...], we_ref[...],
                preferred_element_type=jnp.float32) + be_ref[...]
    deg = jnp.sum(dp_ref[...], axis=1, keepdims=True)
    dinv = lax.rsqrt(1.0 + deg)
    u0_ref[...] = jnp.dot(h, w0_ref[...],
                          preferred_element_type=jnp.float32) * dinv


def _tc_mid_body(u_ref, s_ref, dp_ref, w_ref, g_ref, be_ref, b_ref, out_ref):
    deg = jnp.sum(dp_ref[...], axis=1, keepdims=True)
    dinv = lax.rsqrt(1.0 + deg)
    sc = s_ref[0] + s_ref[1]
    gs = g_ref[...] * _INV_S
    cs = b_ref[...] * gs + be_ref[...]
    h = jnp.maximum(dinv * (u_ref[...] + sc) * gs + cs, 0.0)
    out_ref[...] = jnp.dot(h, w_ref[...],
                           preferred_element_type=jnp.float32) * dinv


def _tc3_body(u_ref, s_ref, dp_ref, g_ref, be_ref, b_ref, batch_ref, md_ref,
              wc1p_ref, wc1m_ref, bc1_ref, wc2_ref, bc2_ref, out_ref,
              sums, cnts):
    i = pl.program_id(0)
    deg = jnp.sum(dp_ref[...], axis=1, keepdims=True)
    dinv = lax.rsqrt(1.0 + deg)
    sc = s_ref[0] + s_ref[1]
    gs = g_ref[...] * _INV_S
    cs = b_ref[...] * gs + be_ref[...]
    h = jnp.maximum(dinv * (u_ref[...] + sc) * gs + cs, 0.0)

    b = batch_ref[0, 0, :]
    onehot = (lax.broadcasted_iota(jnp.int32, (G, _R), 0)
              == b[None, :]).astype(jnp.float32)
    blk_sum = jnp.dot(onehot, h, preferred_element_type=jnp.float32)
    blk_cnt = jnp.sum(onehot, axis=1, keepdims=True)

    @pl.when(i == 0)
    def _init():
        sums[...] = jnp.zeros_like(sums)
        cnts[...] = jnp.zeros_like(cnts)

    sums[...] += blk_sum
    cnts[...] += blk_cnt

    @pl.when(i == _GRID - 1)
    def _final():
        pooled = sums[...] / jnp.maximum(cnts[...], 1.0)
        z = jnp.maximum(
            jnp.dot(pooled, wc1p_ref[...], preferred_element_type=jnp.float32)
            + jnp.dot(md_ref[...], wc1m_ref[...],
                      preferred_element_type=jnp.float32)
            + bc1_ref[...], 0.0)
        logits = lax.dot_general(wc2_ref[...], z, (((0,), (1,)), ((), ())),
                                 preferred_element_type=jnp.float32)
        out_ref[...] = jax.nn.sigmoid(logits + bc2_ref[...])


def _row_spec():
    return pl.BlockSpec((_R, D), lambda i: (i, 0))


def _full(shape):
    nd = len(shape)
    return pl.BlockSpec(shape, lambda i: (0,) * nd)


def kernel(x, edge_index, batch, metadata, W_enc, b_enc, W0, b0, g0, be0,
           W1, b1, g1, be1, Wc1, bc1, Wc2, bc2):
    f32 = jnp.float32
    pad = EPADDED - edge_index.shape[1]
    src2d = jnp.concatenate(
        [edge_index[0], jnp.zeros((pad,), jnp.int32)]).reshape(EPADDED // K, K)
    dst2d = jnp.concatenate(
        [edge_index[1], jnp.full((pad,), DUMP, jnp.int32)]
    ).reshape(EPADDED // K, K)

    _deg, _scatter = _sc_kernels()
    degpart = _deg(dst2d)
    dpT = degpart.reshape(NW, NPAD).T

    dp_spec = pl.BlockSpec((_R, NW), lambda i: (i, 0))
    row_f32 = jax.ShapeDtypeStruct((N, D), f32)

    u0 = pl.pallas_call(
        _tc1_body,
        grid=(_GRID,),
        in_specs=[_row_spec(), _full((D, D)), _full((1, D)), _full((D, D)),
                  dp_spec],
        out_specs=_row_spec(),
        out_shape=row_f32,
    )(x, W_enc, b_enc.reshape(1, D), W0, dpT)

    s0 = _scatter(u0, src2d, dst2d)

    s_spec = pl.BlockSpec((2, _R, D), lambda i: (0, i, 0))
    u1 = pl.pallas_call(
        _tc_mid_body,
        grid=(_GRID,),
        in_specs=[_row_spec(), s_spec, dp_spec, _full((D, D)), _full((1, D)),
                  _full((1, D)), _full((1, D))],
        out_specs=_row_spec(),
        out_shape=row_f32,
    )(u0, s0, dpT, W1, g0.reshape(1, D), be0.reshape(1, D), b0.reshape(1, D))

    s1 = _scatter(u1, src2d, dst2d)

    batch3 = batch.reshape(_GRID, 1, _R)
    out = pl.pallas_call(
        _tc3_body,
        grid=(_GRID,),
        in_specs=[_row_spec(), s_spec, dp_spec, _full((1, D)), _full((1, D)),
                  _full((1, D)), pl.BlockSpec((1, 1, _R), lambda i: (i, 0, 0)),
                  _full((G, MD)), _full((D, G)), _full((MD, G)),
                  _full((1, G)), _full((G, 1)), _full((1, 1))],
        out_specs=_full((1, G)),
        out_shape=jax.ShapeDtypeStruct((1, G), f32),
        scratch_shapes=[pltpu.VMEM((G, D), f32), pltpu.VMEM((G, 1), f32)],
    )(u1, s1, dpT, g1.reshape(1, D), be1.reshape(1, D), b1.reshape(1, D),
      batch3, metadata, Wc1[:D], Wc1[D:], bc1.reshape(1, G),
      Wc2, bc2.reshape(1, 1))

    return out[0]
